# Initial kernel scaffold; baseline (speedup 1.0000x reference)
import functools

import jax
import jax.numpy as jnp
from jax import lax
from jax.experimental import pallas as pl
from jax.experimental.pallas import tpu as pltpu
from jax.experimental.pallas import tpu_sc as plsc

S, R, C = 38, 340, 24
NUM_LAYERS = 4
EPS = 1e-5
B = 256
N = B * (S + R)
E = N * 16
NS = 16
NC = 2
RPT = N // NS
EPT = E // NS
EPW = E // (NS * NC)
KA = 1536
KD = 1512
RB = 8064


def _leaky(v):
    return jnp.where(v >= 0, v, 0.2 * v)


def _deg_body(dst_hbm, zeros_hbm, ones_hbm, out_hbm, dstb, onesb, acc):
    c = lax.axis_index("c")
    s = lax.axis_index("s")
    pltpu.sync_copy(zeros_hbm.at[pl.ds(s * RPT, RPT)], acc.at[pl.ds(s * RPT, RPT)])
    pltpu.sync_copy(ones_hbm, onesb)
    plsc.subcore_barrier()
    base = (c * NS + s) * EPW

    def chunk(i, carry):
        off = base + i * KD
        pltpu.sync_copy(dst_hbm.at[pl.ds(off, KD)], dstb)
        pltpu.sync_copy(onesb, acc.at[dstb], add=True)
        return carry

    lax.fori_loop(0, EPW // KD, chunk, 0)
    plsc.subcore_barrier()
    pltpu.sync_copy(acc.at[pl.ds(s * RPT, RPT)],
                    out_hbm.at[pl.ds(c * N + s * RPT, RPT)])


def _agg_body(src2_hbm, dst_hbm, h2_hbm, out_hbm, srcb, dstb, rows, acc, sem):
    c = lax.axis_index("c")
    s = lax.axis_index("s")
    pltpu.sync_copy(h2_hbm.at[pl.ds(c * N + s * RPT, RPT)],
                    acc.at[pl.ds(s * RPT, RPT)])
    plsc.subcore_barrier()
    base = s * EPT

    def chunk(i, carry):
        off = base + i * KA
        pltpu.sync_copy(src2_hbm.at[pl.ds(c * E + off, KA)], srcb)
        pltpu.sync_copy(dst_hbm.at[pl.ds(off, KA)], dstb)
        pltpu.async_copy(h2_hbm.at[srcb], rows, sem).wait()
        pltpu.sync_copy(rows, acc.at[dstb], add=True)
        return carry

    lax.fori_loop(0, EPT // KA, chunk, 0)
    plsc.subcore_barrier()
    pltpu.sync_copy(acc.at[pl.ds(s * RPT, RPT)],
                    out_hbm.at[pl.ds(c * N + s * RPT, RPT)])


def _sc_mesh():
    return plsc.VectorSubcoreMesh(core_axis_name="c", subcore_axis_name="s")


def _deg_call(dst):
    fn = pl.kernel(
        _deg_body,
        out_type=jax.ShapeDtypeStruct((NC * N,), jnp.float32),
        mesh=_sc_mesh(),
        scratch_types=[
            pltpu.VMEM((KD,), jnp.int32),
            pltpu.VMEM((KD,), jnp.float32),
            pltpu.VMEM_SHARED((N,), jnp.float32),
        ],
    )
    return fn(dst, jnp.zeros((N,), jnp.float32), jnp.ones((KD,), jnp.float32))


def _agg_call(src2, dst, h2):
    fn = pl.kernel(
        _agg_body,
        out_type=jax.ShapeDtypeStruct((NC * N, 16), jnp.float32),
        mesh=_sc_mesh(),
        scratch_types=[
            pltpu.VMEM((KA,), jnp.int32),
            pltpu.VMEM((KA,), jnp.int32),
            pltpu.VMEM((KA, 16), jnp.float32),
            pltpu.VMEM_SHARED((N, 16), jnp.float32),
            pltpu.SemaphoreType.DMA,
        ],
    )
    return fn(src2, dst, h2)


def _prologue_body(sx_ref, rx_ref, bnsw, bnsb, bnrw, bnrb, lsw, lrw,
                   s_out, r_out, ox_out):
    sx = sx_ref[...]
    mu = jnp.mean(sx, axis=0)
    xm = sx - mu
    var = jnp.mean(xm * xm, axis=0)
    sn = xm / jnp.sqrt(var + EPS) * bnsw[0] + bnsb[0]
    s_out[...] = _leaky(jnp.dot(sn, lsw[...].T, preferred_element_type=jnp.float32))

    rx = rx_ref[...]
    rflat = rx[:, :3]
    mu_r = jnp.mean(rflat, axis=0)
    rm = rflat - mu_r
    var_r = jnp.mean(rm * rm, axis=0)
    rn = rm / jnp.sqrt(var_r + EPS) * bnrw[0] + bnrb[0]
    r_out[...] = _leaky(jnp.dot(rn, lrw[...].T, preferred_element_type=jnp.float32))

    ox_out[...] = rx.reshape(B, R, 8)[:, 0, 3:8]


def _prologue_call(s_x, r_x, bn_s_w, bn_s_b, bn_r_w, bn_r_b, lin_s_W, lin_r_W):
    full = lambda shape: pl.BlockSpec(shape, lambda: tuple(0 for _ in shape))
    return pl.pallas_call(
        _prologue_body,
        grid=(),
        in_specs=[
            full((B * S, 4)), full((B * R, 8)),
            full((1, 4)), full((1, 4)), full((1, 3)), full((1, 3)),
            full((C, 4)), full((C, 3)),
        ],
        out_specs=[full((B * S, C)), full((B * R, C)), full((B, 5))],
        out_shape=[
            jax.ShapeDtypeStruct((B * S, C), jnp.float32),
            jax.ShapeDtypeStruct((B * R, C), jnp.float32),
            jax.ShapeDtypeStruct((B, 5), jnp.float32),
        ],
    )(s_x, r_x, bn_s_w.reshape(1, 4), bn_s_b.reshape(1, 4),
      bn_r_w.reshape(1, 3), bn_r_b.reshape(1, 3), lin_s_W, lin_r_W)


def _split_h(h, h2_ref):
    rb = h.shape[0]
    h2_ref[0] = h[:, :16]
    h2_ref[1] = jnp.concatenate([h[:, 16:24], jnp.zeros((rb, 8), jnp.float32)], axis=1)


def _dense0_body(x_ref, degp_ref, w_ref, dinv_ref, h2_ref):
    deg = 1.0 + degp_ref[0, :, 0] + degp_ref[1, :, 0]
    dinv = lax.rsqrt(deg)
    dinv_ref[:, 0] = dinv
    h = jnp.dot(x_ref[...], w_ref[...].T, preferred_element_type=jnp.float32)
    _split_h(h * dinv[:, None], h2_ref)


def _dense0_call(x0, degp, W0):
    grid = (N // RB,)
    return pl.pallas_call(
        _dense0_body,
        grid=grid,
        in_specs=[
            pl.BlockSpec((RB, C), lambda i: (i, 0)),
            pl.BlockSpec((2, RB, 1), lambda i: (0, i, 0)),
            pl.BlockSpec((C, C), lambda i: (0, 0)),
        ],
        out_specs=[
            pl.BlockSpec((RB, 1), lambda i: (i, 0)),
            pl.BlockSpec((2, RB, 16), lambda i: (0, i, 0)),
        ],
        out_shape=[
            jax.ShapeDtypeStruct((N, 1), jnp.float32),
            jax.ShapeDtypeStruct((2, N, 16), jnp.float32),
        ],
    )(x0, degp, W0)


def _dense_body(x_ref, agg_ref, dinv_ref, b_ref, w_ref, xn_ref, h2_ref):
    dinv = dinv_ref[:, 0]
    agg = jnp.concatenate([agg_ref[0], agg_ref[1, :, :8]], axis=1)
    xn = x_ref[...] + _leaky(dinv[:, None] * agg + b_ref[0])
    xn_ref[...] = xn
    h = jnp.dot(xn, w_ref[...].T, preferred_element_type=jnp.float32)
    _split_h(h * dinv[:, None], h2_ref)


def _dense_call(x, agg2, dinv, b, Wnext):
    grid = (N // RB,)
    return pl.pallas_call(
        _dense_body,
        grid=grid,
        in_specs=[
            pl.BlockSpec((RB, C), lambda i: (i, 0)),
            pl.BlockSpec((2, RB, 16), lambda i: (0, i, 0)),
            pl.BlockSpec((RB, 1), lambda i: (i, 0)),
            pl.BlockSpec((1, C), lambda i: (0, 0)),
            pl.BlockSpec((C, C), lambda i: (0, 0)),
        ],
        out_specs=[
            pl.BlockSpec((RB, C), lambda i: (i, 0)),
            pl.BlockSpec((2, RB, 16), lambda i: (0, i, 0)),
        ],
        out_shape=[
            jax.ShapeDtypeStruct((N, C), jnp.float32),
            jax.ShapeDtypeStruct((2, N, 16), jnp.float32),
        ],
    )(x, agg2, dinv, b.reshape(1, C), Wnext)


def _dense_last_body(x_ref, agg_ref, dinv_ref, b_ref, xn_ref):
    dinv = dinv_ref[:, 0]
    agg = jnp.concatenate([agg_ref[0], agg_ref[1, :, :8]], axis=1)
    xn_ref[...] = x_ref[...] + _leaky(dinv[:, None] * agg + b_ref[0])


def _dense_last_call(x, agg2, dinv, b):
    grid = (N // RB,)
    return pl.pallas_call(
        _dense_last_body,
        grid=grid,
        in_specs=[
            pl.BlockSpec((RB, C), lambda i: (i, 0)),
            pl.BlockSpec((2, RB, 16), lambda i: (0, i, 0)),
            pl.BlockSpec((RB, 1), lambda i: (i, 0)),
            pl.BlockSpec((1, C), lambda i: (0, 0)),
        ],
        out_specs=pl.BlockSpec((RB, C), lambda i: (i, 0)),
        out_shape=jax.ShapeDtypeStruct((N, C), jnp.float32),
    )(x, agg2, dinv, b.reshape(1, C))


def _readout_body(x_ref, ox_ref, lrw, lrb, w1, b1, w2, b2, out_ref):
    xg = jnp.mean(x_ref[...], axis=2)
    logits = jnp.dot(xg, lrw[...].T, preferred_element_type=jnp.float32) + lrb[0]
    exl = jnp.exp(logits)
    p = exl / (jnp.sum(exl, axis=1, keepdims=True) + 1.0)
    o = _leaky(jnp.dot(ox_ref[...], w1[...].T, preferred_element_type=jnp.float32) + b1[0])
    o = jnp.dot(o, w2[...].T, preferred_element_type=jnp.float32) + b2[0]
    out_ref[...] = p * jnp.exp(o)


def _readout_call(x4, o_x, linr_W, linr_b, lino_W1, lino_b1, lino_W2, lino_b2):
    BB = 32
    grid = (B // BB,)
    return pl.pallas_call(
        _readout_body,
        grid=grid,
        in_specs=[
            pl.BlockSpec((BB, S + R, C), lambda i: (i, 0, 0)),
            pl.BlockSpec((BB, 5), lambda i: (i, 0)),
            pl.BlockSpec((7, S + R), lambda i: (0, 0)),
            pl.BlockSpec((1, 7), lambda i: (0, 0)),
            pl.BlockSpec((C, 5), lambda i: (0, 0)),
            pl.BlockSpec((1, C), lambda i: (0, 0)),
            pl.BlockSpec((7, C), lambda i: (0, 0)),
            pl.BlockSpec((1, 7), lambda i: (0, 0)),
        ],
        out_specs=pl.BlockSpec((BB, 7), lambda i: (i, 0)),
        out_shape=jax.ShapeDtypeStruct((B, 7), jnp.float32),
    )(x4.reshape(B, S + R, C), o_x, linr_W, linr_b.reshape(1, 7),
      lino_W1, lino_b1.reshape(1, C), lino_W2, lino_b2.reshape(1, 7))


def kernel(s_x, r_x, edge_index, bn_s_w, bn_s_b, bn_r_w, bn_r_b, lin_s_W,
           lin_r_W, conv_W, conv_b, linr_W, linr_b, lino_W1, lino_b1,
           lino_W2, lino_b2):
    src = edge_index[0]
    dst = edge_index[1]
    src2 = jnp.concatenate([src, src + N])

    degp = _deg_call(dst)
    s_emb, r_emb, o_x = _prologue_call(
        s_x, r_x, bn_s_w, bn_s_b, bn_r_w, bn_r_b, lin_s_W, lin_r_W)
    x = jnp.concatenate(
        [s_emb.reshape(B, S, C), r_emb.reshape(B, R, C)], axis=1).reshape(N, C)

    dinv, h2 = _dense0_call(x, degp.reshape(2, N, 1), conv_W[0])
    for l in range(NUM_LAYERS):
        agg2 = _agg_call(src2, dst, h2.reshape(2 * N, 16))
        if l < NUM_LAYERS - 1:
            x, h2 = _dense_call(x, agg2.reshape(2, N, 16), dinv,
                                conv_b[l], conv_W[l + 1])
        else:
            x = _dense_last_call(x, agg2.reshape(2, N, 16), dinv, conv_b[l])

    return _readout_call(x, o_x, linr_W, linr_b,
                         lino_W1, lino_b1, lino_W2, lino_b2)


# speedup vs baseline: 21.2493x; 21.2493x over previous
import functools

import jax
import jax.numpy as jnp
from jax import lax
from jax.experimental import pallas as pl
from jax.experimental.pallas import tpu as pltpu
from jax.experimental.pallas import tpu_sc as plsc

S, R, C = 38, 340, 24
NUM_LAYERS = 4
EPS = 1e-5
B = 256
N = B * (S + R)
E = N * 16
NS = 16
NC = 2
RPT = N // NS
EPT = E // NS
EPW = E // (NS * NC)
KA = 1536
KD = 1512
RB = 2016


def _leaky(v):
    return jnp.where(v >= 0, v, 0.2 * v)


RCH = 1512


def _deg_body(dst_hbm, zeros_hbm, ones_hbm, out_hbm, dstb, onesb, zbuf, acc):
    c = lax.axis_index("c")
    s = lax.axis_index("s")
    pltpu.sync_copy(zeros_hbm, zbuf)
    pltpu.sync_copy(ones_hbm, onesb)
    for j in range(RPT // RCH):
        pltpu.sync_copy(zbuf, acc.at[pl.ds(s * RPT + j * RCH, RCH)])
    plsc.subcore_barrier()
    base = (c * NS + s) * EPW

    def chunk(i, carry):
        off = base + i * KD
        pltpu.sync_copy(dst_hbm.at[pl.ds(off, KD)], dstb)
        pltpu.sync_copy(onesb, acc.at[dstb], add=True)
        return carry

    lax.fori_loop(0, EPW // KD, chunk, 0)
    plsc.subcore_barrier()
    for j in range(RPT // RCH):
        pltpu.sync_copy(acc.at[pl.ds(s * RPT + j * RCH, RCH)], zbuf)
        pltpu.sync_copy(zbuf, out_hbm.at[pl.ds(c * N + s * RPT + j * RCH, RCH)])


def _agg_body(src2_hbm, dst_hbm, h2_hbm, out_hbm, srcb, dstb, rows, acc, sem):
    c = lax.axis_index("c")
    s = lax.axis_index("s")
    for j in range(RPT // RCH):
        pltpu.sync_copy(h2_hbm.at[pl.ds(c * N + s * RPT + j * RCH, RCH)],
                        rows.at[pl.ds(0, RCH)])
        pltpu.sync_copy(rows.at[pl.ds(0, RCH)],
                        acc.at[pl.ds(s * RPT + j * RCH, RCH)])
    plsc.subcore_barrier()
    base = s * EPT

    def chunk(i, carry):
        off = base + i * KA
        pltpu.sync_copy(src2_hbm.at[pl.ds(c * E + off, KA)], srcb)
        pltpu.sync_copy(dst_hbm.at[pl.ds(off, KA)], dstb)
        pltpu.async_copy(h2_hbm.at[srcb], rows, sem).wait()
        pltpu.sync_copy(rows, acc.at[dstb], add=True)
        return carry

    lax.fori_loop(0, EPT // KA, chunk, 0)
    plsc.subcore_barrier()
    for j in range(RPT // RCH):
        pltpu.sync_copy(acc.at[pl.ds(s * RPT + j * RCH, RCH)],
                        rows.at[pl.ds(0, RCH)])
        pltpu.sync_copy(rows.at[pl.ds(0, RCH)],
                        out_hbm.at[pl.ds(c * N + s * RPT + j * RCH, RCH)])


def _sc_mesh():
    return plsc.VectorSubcoreMesh(core_axis_name="c", subcore_axis_name="s")


def _deg_call(dst):
    fn = pl.kernel(
        _deg_body,
        out_type=jax.ShapeDtypeStruct((NC * N,), jnp.float32),
        mesh=_sc_mesh(),
        scratch_types=[
            pltpu.VMEM((KD,), jnp.int32),
            pltpu.VMEM((KD,), jnp.float32),
            pltpu.VMEM((RCH,), jnp.float32),
            pltpu.VMEM_SHARED((N,), jnp.float32),
        ],
        compiler_params=pltpu.CompilerParams(use_tc_tiling_on_sc=False),
    )
    return fn(dst, jnp.zeros((RCH,), jnp.float32), jnp.ones((KD,), jnp.float32))


def _agg_call(src2, dst, h2):
    fn = pl.kernel(
        _agg_body,
        out_type=jax.ShapeDtypeStruct((NC * N, 16), jnp.float32),
        mesh=_sc_mesh(),
        scratch_types=[
            pltpu.VMEM((KA,), jnp.int32),
            pltpu.VMEM((KA,), jnp.int32),
            pltpu.VMEM((KA, 16), jnp.float32),
            pltpu.VMEM_SHARED((N, 16), jnp.float32),
            pltpu.SemaphoreType.DMA,
        ],
        compiler_params=pltpu.CompilerParams(use_tc_tiling_on_sc=False),
    )
    return fn(src2, dst, h2)


PG = 16
SBK = B * S // PG
RBK = B * R // PG


def _stats_body(sx_ref, rx_ref, ss_ref, rs_ref):
    i = pl.program_id(0)
    sx = sx_ref[...]
    rflat = rx_ref[...][:, :3]

    @pl.when(i == 0)
    def _():
        ss_ref[...] = jnp.zeros_like(ss_ref)
        rs_ref[...] = jnp.zeros_like(rs_ref)

    ss_ref[0, :] += jnp.sum(sx, axis=0)
    ss_ref[1, :] += jnp.sum(sx * sx, axis=0)
    rs_ref[0, :] += jnp.sum(rflat, axis=0)
    rs_ref[1, :] += jnp.sum(rflat * rflat, axis=0)


def _stats_call(s_x, r_x):
    return pl.pallas_call(
        _stats_body,
        grid=(PG,),
        in_specs=[
            pl.BlockSpec((SBK, 4), lambda i: (i, 0)),
            pl.BlockSpec((RBK, 8), lambda i: (i, 0)),
        ],
        out_specs=[
            pl.BlockSpec((2, 4), lambda i: (0, 0)),
            pl.BlockSpec((2, 3), lambda i: (0, 0)),
        ],
        out_shape=[
            jax.ShapeDtypeStruct((2, 4), jnp.float32),
            jax.ShapeDtypeStruct((2, 3), jnp.float32),
        ],
    )(s_x, r_x)


def _prologue_body(sx_ref, rx_ref, ss_ref, rs_ref, bnsw, bnsb, bnrw, bnrb,
                   lsw, lrw, s_out, r_out, ox_out):
    ns = float(B * S)
    mu = ss_ref[0, :] / ns
    var = ss_ref[1, :] / ns - mu * mu
    sn = (sx_ref[...] - mu) / jnp.sqrt(var + EPS) * bnsw[0] + bnsb[0]
    s_out[...] = _leaky(jnp.dot(sn, lsw[...].T, preferred_element_type=jnp.float32))

    rx = rx_ref[...]
    nr = float(B * R)
    mu_r = rs_ref[0, :] / nr
    var_r = rs_ref[1, :] / nr - mu_r * mu_r
    rn = (rx[:, :3] - mu_r) / jnp.sqrt(var_r + EPS) * bnrw[0] + bnrb[0]
    r_out[...] = _leaky(jnp.dot(rn, lrw[...].T, preferred_element_type=jnp.float32))

    ox_out[...] = rx.reshape(RBK // R, R, 8)[:, 0, 3:8]


def _prologue_call(s_x, r_x, bn_s_w, bn_s_b, bn_r_w, bn_r_b, lin_s_W, lin_r_W):
    sstats, rstats = _stats_call(s_x, r_x)
    return pl.pallas_call(
        _prologue_body,
        grid=(PG,),
        in_specs=[
            pl.BlockSpec((SBK, 4), lambda i: (i, 0)),
            pl.BlockSpec((RBK, 8), lambda i: (i, 0)),
            pl.BlockSpec((2, 4), lambda i: (0, 0)),
            pl.BlockSpec((2, 3), lambda i: (0, 0)),
            pl.BlockSpec((1, 4), lambda i: (0, 0)),
            pl.BlockSpec((1, 4), lambda i: (0, 0)),
            pl.BlockSpec((1, 3), lambda i: (0, 0)),
            pl.BlockSpec((1, 3), lambda i: (0, 0)),
            pl.BlockSpec((C, 4), lambda i: (0, 0)),
            pl.BlockSpec((C, 3), lambda i: (0, 0)),
        ],
        out_specs=[
            pl.BlockSpec((SBK, C), lambda i: (i, 0)),
            pl.BlockSpec((RBK, C), lambda i: (i, 0)),
            pl.BlockSpec((RBK // R, 5), lambda i: (i, 0)),
        ],
        out_shape=[
            jax.ShapeDtypeStruct((B * S, C), jnp.float32),
            jax.ShapeDtypeStruct((B * R, C), jnp.float32),
            jax.ShapeDtypeStruct((B, 5), jnp.float32),
        ],
    )(s_x, r_x, sstats, rstats, bn_s_w.reshape(1, 4), bn_s_b.reshape(1, 4),
      bn_r_w.reshape(1, 3), bn_r_b.reshape(1, 3), lin_s_W, lin_r_W)


def _split_h(h, h2_ref):
    rb = h.shape[0]
    h2_ref[0] = h[:, :16]
    h2_ref[1] = jnp.concatenate([h[:, 16:24], jnp.zeros((rb, 8), jnp.float32)], axis=1)


def _dense0_body(x_ref, degp_ref, w_ref, dinv_ref, h2_ref):
    deg = 1.0 + degp_ref[0, :, 0] + degp_ref[1, :, 0]
    dinv = lax.rsqrt(deg)
    dinv_ref[:, 0] = dinv
    h = jnp.dot(x_ref[...], w_ref[...].T, preferred_element_type=jnp.float32)
    _split_h(h * dinv[:, None], h2_ref)


def _dense0_call(x0, degp, W0):
    grid = (N // RB,)
    return pl.pallas_call(
        _dense0_body,
        grid=grid,
        in_specs=[
            pl.BlockSpec((RB, C), lambda i: (i, 0)),
            pl.BlockSpec((2, RB, 1), lambda i: (0, i, 0)),
            pl.BlockSpec((C, C), lambda i: (0, 0)),
        ],
        out_specs=[
            pl.BlockSpec((RB, 1), lambda i: (i, 0)),
            pl.BlockSpec((2, RB, 16), lambda i: (0, i, 0)),
        ],
        out_shape=[
            jax.ShapeDtypeStruct((N, 1), jnp.float32),
            jax.ShapeDtypeStruct((2, N, 16), jnp.float32),
        ],
    )(x0, degp, W0)


def _dense_body(x_ref, agg_ref, dinv_ref, b_ref, w_ref, xn_ref, h2_ref):
    dinv = dinv_ref[:, 0]
    agg = jnp.concatenate([agg_ref[0], agg_ref[1, :, :8]], axis=1)
    xn = x_ref[...] + _leaky(dinv[:, None] * agg + b_ref[0])
    xn_ref[...] = xn
    h = jnp.dot(xn, w_ref[...].T, preferred_element_type=jnp.float32)
    _split_h(h * dinv[:, None], h2_ref)


def _dense_call(x, agg2, dinv, b, Wnext):
    grid = (N // RB,)
    return pl.pallas_call(
        _dense_body,
        grid=grid,
        in_specs=[
            pl.BlockSpec((RB, C), lambda i: (i, 0)),
            pl.BlockSpec((2, RB, 16), lambda i: (0, i, 0)),
            pl.BlockSpec((RB, 1), lambda i: (i, 0)),
            pl.BlockSpec((1, C), lambda i: (0, 0)),
            pl.BlockSpec((C, C), lambda i: (0, 0)),
        ],
        out_specs=[
            pl.BlockSpec((RB, C), lambda i: (i, 0)),
            pl.BlockSpec((2, RB, 16), lambda i: (0, i, 0)),
        ],
        out_shape=[
            jax.ShapeDtypeStruct((N, C), jnp.float32),
            jax.ShapeDtypeStruct((2, N, 16), jnp.float32),
        ],
    )(x, agg2, dinv, b.reshape(1, C), Wnext)


def _dense_last_body(x_ref, agg_ref, dinv_ref, b_ref, xn_ref):
    dinv = dinv_ref[:, 0]
    agg = jnp.concatenate([agg_ref[0], agg_ref[1, :, :8]], axis=1)
    xn_ref[...] = x_ref[...] + _leaky(dinv[:, None] * agg + b_ref[0])


def _dense_last_call(x, agg2, dinv, b):
    grid = (N // RB,)
    return pl.pallas_call(
        _dense_last_body,
        grid=grid,
        in_specs=[
            pl.BlockSpec((RB, C), lambda i: (i, 0)),
            pl.BlockSpec((2, RB, 16), lambda i: (0, i, 0)),
            pl.BlockSpec((RB, 1), lambda i: (i, 0)),
            pl.BlockSpec((1, C), lambda i: (0, 0)),
        ],
        out_specs=pl.BlockSpec((RB, C), lambda i: (i, 0)),
        out_shape=jax.ShapeDtypeStruct((N, C), jnp.float32),
    )(x, agg2, dinv, b.reshape(1, C))


def _readout_body(x_ref, ox_ref, lrw, lrb, w1, b1, w2, b2, out_ref):
    xg = jnp.mean(x_ref[...], axis=2)
    logits = jnp.dot(xg, lrw[...].T, preferred_element_type=jnp.float32) + lrb[0]
    exl = jnp.exp(logits)
    p = exl / (jnp.sum(exl, axis=1, keepdims=True) + 1.0)
    o = _leaky(jnp.dot(ox_ref[...], w1[...].T, preferred_element_type=jnp.float32) + b1[0])
    o = jnp.dot(o, w2[...].T, preferred_element_type=jnp.float32) + b2[0]
    out_ref[...] = p * jnp.exp(o)


def _readout_call(x4, o_x, linr_W, linr_b, lino_W1, lino_b1, lino_W2, lino_b2):
    BB = 32
    grid = (B // BB,)
    return pl.pallas_call(
        _readout_body,
        grid=grid,
        in_specs=[
            pl.BlockSpec((BB, S + R, C), lambda i: (i, 0, 0)),
            pl.BlockSpec((BB, 5), lambda i: (i, 0)),
            pl.BlockSpec((7, S + R), lambda i: (0, 0)),
            pl.BlockSpec((1, 7), lambda i: (0, 0)),
            pl.BlockSpec((C, 5), lambda i: (0, 0)),
            pl.BlockSpec((1, C), lambda i: (0, 0)),
            pl.BlockSpec((7, C), lambda i: (0, 0)),
            pl.BlockSpec((1, 7), lambda i: (0, 0)),
        ],
        out_specs=pl.BlockSpec((BB, 7), lambda i: (i, 0)),
        out_shape=jax.ShapeDtypeStruct((B, 7), jnp.float32),
    )(x4.reshape(B, S + R, C), o_x, linr_W, linr_b.reshape(1, 7),
      lino_W1, lino_b1.reshape(1, C), lino_W2, lino_b2.reshape(1, 7))


def kernel(s_x, r_x, edge_index, bn_s_w, bn_s_b, bn_r_w, bn_r_b, lin_s_W,
           lin_r_W, conv_W, conv_b, linr_W, linr_b, lino_W1, lino_b1,
           lino_W2, lino_b2):
    src = edge_index[0]
    dst = edge_index[1]
    src2 = jnp.concatenate([src, src + N])

    degp = _deg_call(dst)
    s_emb, r_emb, o_x = _prologue_call(
        s_x, r_x, bn_s_w, bn_s_b, bn_r_w, bn_r_b, lin_s_W, lin_r_W)
    x = jnp.concatenate(
        [s_emb.reshape(B, S, C), r_emb.reshape(B, R, C)], axis=1).reshape(N, C)

    dinv, h2 = _dense0_call(x, degp.reshape(2, N, 1), conv_W[0])
    for l in range(NUM_LAYERS):
        agg2 = _agg_call(src2, dst, h2.reshape(2 * N, 16))
        if l < NUM_LAYERS - 1:
            x, h2 = _dense_call(x, agg2.reshape(2, N, 16), dinv,
                                conv_b[l], conv_W[l + 1])
        else:
            x = _dense_last_call(x, agg2.reshape(2, N, 16), dinv, conv_b[l])

    return _readout_call(x, o_x, linr_W, linr_b,
                         lino_W1, lino_b1, lino_W2, lino_b2)


# speedup vs baseline: 24.4272x; 1.1496x over previous
import functools

import jax
import jax.numpy as jnp
from jax import lax
from jax.experimental import pallas as pl
from jax.experimental.pallas import tpu as pltpu
from jax.experimental.pallas import tpu_sc as plsc

S, R, C = 38, 340, 24
NUM_LAYERS = 4
EPS = 1e-5
B = 256
N = B * (S + R)
E = N * 16
NS = 16
NC = 2
RPT = N // NS
EPT = E // NS
EPW = E // (NS * NC)
KA = 864
KD = 1512
RB = 2016


def _leaky(v):
    return jnp.where(v >= 0, v, 0.2 * v)


RCH = 864


def _deg_body(dst_hbm, zeros_hbm, ones_hbm, out_hbm, dstb, onesb, zbuf, acc):
    c = lax.axis_index("c")
    s = lax.axis_index("s")
    pltpu.sync_copy(zeros_hbm, zbuf)
    pltpu.sync_copy(ones_hbm, onesb)
    for j in range(RPT // RCH):
        pltpu.sync_copy(zbuf, acc.at[pl.ds(s * RPT + j * RCH, RCH)])
    plsc.subcore_barrier()
    base = (c * NS + s) * EPW

    def chunk(i, carry):
        off = base + i * KD
        pltpu.sync_copy(dst_hbm.at[pl.ds(off, KD)], dstb)
        pltpu.sync_copy(onesb, acc.at[dstb], add=True)
        return carry

    lax.fori_loop(0, EPW // KD, chunk, 0)
    plsc.subcore_barrier()
    for j in range(RPT // RCH):
        pltpu.sync_copy(acc.at[pl.ds(s * RPT + j * RCH, RCH)], zbuf)
        pltpu.sync_copy(zbuf, out_hbm.at[pl.ds(c * N + s * RPT + j * RCH, RCH)])


def _agg_body(src2_hbm, dst_hbm, h2_hbm, out_hbm,
              srcA, dstA, rowsA, srcB, dstB, rowsB, semA, semB, acc):
    c = lax.axis_index("c")
    s = lax.axis_index("s")
    for j in range(RPT // RCH):
        pltpu.sync_copy(h2_hbm.at[pl.ds(c * N + s * RPT + j * RCH, RCH)],
                        rowsA.at[pl.ds(0, RCH)])
        pltpu.sync_copy(rowsA.at[pl.ds(0, RCH)],
                        acc.at[pl.ds(s * RPT + j * RCH, RCH)])
    plsc.subcore_barrier()
    base = s * EPT
    npair = EPT // KA // 2

    pltpu.sync_copy(src2_hbm.at[pl.ds(c * E + base, KA)], srcA)
    pltpu.sync_copy(dst_hbm.at[pl.ds(base, KA)], dstA)
    pltpu.async_copy(h2_hbm.at[srcA], rowsA, semA)

    def pair(j, carry):
        offB = base + (2 * j + 1) * KA
        pltpu.sync_copy(src2_hbm.at[pl.ds(c * E + offB, KA)], srcB)
        pltpu.sync_copy(dst_hbm.at[pl.ds(offB, KA)], dstB)
        pltpu.async_copy(h2_hbm.at[srcB], rowsB, semB)
        pltpu.make_async_copy(h2_hbm.at[srcA], rowsA, semA).wait()
        pltpu.sync_copy(rowsA, acc.at[dstA], add=True)

        @pl.when(j < npair - 1)
        def _():
            offA = base + (2 * j + 2) * KA
            pltpu.sync_copy(src2_hbm.at[pl.ds(c * E + offA, KA)], srcA)
            pltpu.sync_copy(dst_hbm.at[pl.ds(offA, KA)], dstA)
            pltpu.async_copy(h2_hbm.at[srcA], rowsA, semA)

        pltpu.make_async_copy(h2_hbm.at[srcB], rowsB, semB).wait()
        pltpu.sync_copy(rowsB, acc.at[dstB], add=True)
        return carry

    lax.fori_loop(0, npair, pair, 0)
    plsc.subcore_barrier()
    for j in range(RPT // RCH):
        pltpu.sync_copy(acc.at[pl.ds(s * RPT + j * RCH, RCH)],
                        rowsA.at[pl.ds(0, RCH)])
        pltpu.sync_copy(rowsA.at[pl.ds(0, RCH)],
                        out_hbm.at[pl.ds(c * N + s * RPT + j * RCH, RCH)])


def _sc_mesh():
    return plsc.VectorSubcoreMesh(core_axis_name="c", subcore_axis_name="s")


def _deg_call(dst):
    fn = pl.kernel(
        _deg_body,
        out_type=jax.ShapeDtypeStruct((NC * N,), jnp.float32),
        mesh=_sc_mesh(),
        scratch_types=[
            pltpu.VMEM((KD,), jnp.int32),
            pltpu.VMEM((KD,), jnp.float32),
            pltpu.VMEM((RCH,), jnp.float32),
            pltpu.VMEM_SHARED((N,), jnp.float32),
        ],
        compiler_params=pltpu.CompilerParams(use_tc_tiling_on_sc=False),
    )
    return fn(dst, jnp.zeros((RCH,), jnp.float32), jnp.ones((KD,), jnp.float32))


def _agg_call(src2, dst, h2):
    fn = pl.kernel(
        _agg_body,
        out_type=jax.ShapeDtypeStruct((NC * N, 16), jnp.float32),
        mesh=_sc_mesh(),
        scratch_types=[
            pltpu.VMEM((KA,), jnp.int32),
            pltpu.VMEM((KA,), jnp.int32),
            pltpu.VMEM((KA, 16), jnp.float32),
            pltpu.VMEM((KA,), jnp.int32),
            pltpu.VMEM((KA,), jnp.int32),
            pltpu.VMEM((KA, 16), jnp.float32),
            pltpu.SemaphoreType.DMA,
            pltpu.SemaphoreType.DMA,
            pltpu.VMEM_SHARED((N, 16), jnp.float32),
        ],
        compiler_params=pltpu.CompilerParams(use_tc_tiling_on_sc=False),
    )
    return fn(src2, dst, h2)


PG = 16
SBK = B * S // PG
RBK = B * R // PG


def _stats_body(sx_ref, rx_ref, ss_ref, rs_ref):
    i = pl.program_id(0)
    sx = sx_ref[...]
    rflat = rx_ref[...][:, :3]

    @pl.when(i == 0)
    def _():
        ss_ref[...] = jnp.zeros_like(ss_ref)
        rs_ref[...] = jnp.zeros_like(rs_ref)

    ss_ref[0, :] += jnp.sum(sx, axis=0)
    ss_ref[1, :] += jnp.sum(sx * sx, axis=0)
    rs_ref[0, :] += jnp.sum(rflat, axis=0)
    rs_ref[1, :] += jnp.sum(rflat * rflat, axis=0)


def _stats_call(s_x, r_x):
    return pl.pallas_call(
        _stats_body,
        grid=(PG,),
        in_specs=[
            pl.BlockSpec((SBK, 4), lambda i: (i, 0)),
            pl.BlockSpec((RBK, 8), lambda i: (i, 0)),
        ],
        out_specs=[
            pl.BlockSpec((2, 4), lambda i: (0, 0)),
            pl.BlockSpec((2, 3), lambda i: (0, 0)),
        ],
        out_shape=[
            jax.ShapeDtypeStruct((2, 4), jnp.float32),
            jax.ShapeDtypeStruct((2, 3), jnp.float32),
        ],
    )(s_x, r_x)


def _prologue_body(sx_ref, rx_ref, ss_ref, rs_ref, bnsw, bnsb, bnrw, bnrb,
                   lsw, lrw, s_out, r_out, ox_out):
    ns = float(B * S)
    mu = ss_ref[0, :] / ns
    var = ss_ref[1, :] / ns - mu * mu
    sn = (sx_ref[...] - mu) / jnp.sqrt(var + EPS) * bnsw[0] + bnsb[0]
    s_out[...] = _leaky(jnp.dot(sn, lsw[...].T, preferred_element_type=jnp.float32))

    rx = rx_ref[...]
    nr = float(B * R)
    mu_r = rs_ref[0, :] / nr
    var_r = rs_ref[1, :] / nr - mu_r * mu_r
    rn = (rx[:, :3] - mu_r) / jnp.sqrt(var_r + EPS) * bnrw[0] + bnrb[0]
    r_out[...] = _leaky(jnp.dot(rn, lrw[...].T, preferred_element_type=jnp.float32))

    ox_out[...] = rx.reshape(RBK // R, R, 8)[:, 0, 3:8]


def _prologue_call(s_x, r_x, bn_s_w, bn_s_b, bn_r_w, bn_r_b, lin_s_W, lin_r_W):
    sstats, rstats = _stats_call(s_x, r_x)
    return pl.pallas_call(
        _prologue_body,
        grid=(PG,),
        in_specs=[
            pl.BlockSpec((SBK, 4), lambda i: (i, 0)),
            pl.BlockSpec((RBK, 8), lambda i: (i, 0)),
            pl.BlockSpec((2, 4), lambda i: (0, 0)),
            pl.BlockSpec((2, 3), lambda i: (0, 0)),
            pl.BlockSpec((1, 4), lambda i: (0, 0)),
            pl.BlockSpec((1, 4), lambda i: (0, 0)),
            pl.BlockSpec((1, 3), lambda i: (0, 0)),
            pl.BlockSpec((1, 3), lambda i: (0, 0)),
            pl.BlockSpec((C, 4), lambda i: (0, 0)),
            pl.BlockSpec((C, 3), lambda i: (0, 0)),
        ],
        out_specs=[
            pl.BlockSpec((SBK, C), lambda i: (i, 0)),
            pl.BlockSpec((RBK, C), lambda i: (i, 0)),
            pl.BlockSpec((RBK // R, 5), lambda i: (i, 0)),
        ],
        out_shape=[
            jax.ShapeDtypeStruct((B * S, C), jnp.float32),
            jax.ShapeDtypeStruct((B * R, C), jnp.float32),
            jax.ShapeDtypeStruct((B, 5), jnp.float32),
        ],
    )(s_x, r_x, sstats, rstats, bn_s_w.reshape(1, 4), bn_s_b.reshape(1, 4),
      bn_r_w.reshape(1, 3), bn_r_b.reshape(1, 3), lin_s_W, lin_r_W)


def _split_h(h, h2_ref):
    rb = h.shape[0]
    h2_ref[0] = h[:, :16]
    h2_ref[1] = jnp.concatenate([h[:, 16:24], jnp.zeros((rb, 8), jnp.float32)], axis=1)


def _dense0_body(x_ref, degp_ref, w_ref, dinv_ref, h2_ref):
    deg = 1.0 + degp_ref[0, :, 0] + degp_ref[1, :, 0]
    dinv = lax.rsqrt(deg)
    dinv_ref[:, 0] = dinv
    h = jnp.dot(x_ref[...], w_ref[...].T, preferred_element_type=jnp.float32)
    _split_h(h * dinv[:, None], h2_ref)


def _dense0_call(x0, degp, W0):
    grid = (N // RB,)
    return pl.pallas_call(
        _dense0_body,
        grid=grid,
        in_specs=[
            pl.BlockSpec((RB, C), lambda i: (i, 0)),
            pl.BlockSpec((2, RB, 1), lambda i: (0, i, 0)),
            pl.BlockSpec((C, C), lambda i: (0, 0)),
        ],
        out_specs=[
            pl.BlockSpec((RB, 1), lambda i: (i, 0)),
            pl.BlockSpec((2, RB, 16), lambda i: (0, i, 0)),
        ],
        out_shape=[
            jax.ShapeDtypeStruct((N, 1), jnp.float32),
            jax.ShapeDtypeStruct((2, N, 16), jnp.float32),
        ],
    )(x0, degp, W0)


def _dense_body(x_ref, agg_ref, dinv_ref, b_ref, w_ref, xn_ref, h2_ref):
    dinv = dinv_ref[:, 0]
    agg = jnp.concatenate([agg_ref[0], agg_ref[1, :, :8]], axis=1)
    xn = x_ref[...] + _leaky(dinv[:, None] * agg + b_ref[0])
    xn_ref[...] = xn
    h = jnp.dot(xn, w_ref[...].T, preferred_element_type=jnp.float32)
    _split_h(h * dinv[:, None], h2_ref)


def _dense_call(x, agg2, dinv, b, Wnext):
    grid = (N // RB,)
    return pl.pallas_call(
        _dense_body,
        grid=grid,
        in_specs=[
            pl.BlockSpec((RB, C), lambda i: (i, 0)),
            pl.BlockSpec((2, RB, 16), lambda i: (0, i, 0)),
            pl.BlockSpec((RB, 1), lambda i: (i, 0)),
            pl.BlockSpec((1, C), lambda i: (0, 0)),
            pl.BlockSpec((C, C), lambda i: (0, 0)),
        ],
        out_specs=[
            pl.BlockSpec((RB, C), lambda i: (i, 0)),
            pl.BlockSpec((2, RB, 16), lambda i: (0, i, 0)),
        ],
        out_shape=[
            jax.ShapeDtypeStruct((N, C), jnp.float32),
            jax.ShapeDtypeStruct((2, N, 16), jnp.float32),
        ],
    )(x, agg2, dinv, b.reshape(1, C), Wnext)


def _dense_last_body(x_ref, agg_ref, dinv_ref, b_ref, xn_ref):
    dinv = dinv_ref[:, 0]
    agg = jnp.concatenate([agg_ref[0], agg_ref[1, :, :8]], axis=1)
    xn_ref[...] = x_ref[...] + _leaky(dinv[:, None] * agg + b_ref[0])


def _dense_last_call(x, agg2, dinv, b):
    grid = (N // RB,)
    return pl.pallas_call(
        _dense_last_body,
        grid=grid,
        in_specs=[
            pl.BlockSpec((RB, C), lambda i: (i, 0)),
            pl.BlockSpec((2, RB, 16), lambda i: (0, i, 0)),
            pl.BlockSpec((RB, 1), lambda i: (i, 0)),
            pl.BlockSpec((1, C), lambda i: (0, 0)),
        ],
        out_specs=pl.BlockSpec((RB, C), lambda i: (i, 0)),
        out_shape=jax.ShapeDtypeStruct((N, C), jnp.float32),
    )(x, agg2, dinv, b.reshape(1, C))


def _readout_body(x_ref, ox_ref, lrw, lrb, w1, b1, w2, b2, out_ref):
    xg = jnp.mean(x_ref[...], axis=2)
    logits = jnp.dot(xg, lrw[...].T, preferred_element_type=jnp.float32) + lrb[0]
    exl = jnp.exp(logits)
    p = exl / (jnp.sum(exl, axis=1, keepdims=True) + 1.0)
    o = _leaky(jnp.dot(ox_ref[...], w1[...].T, preferred_element_type=jnp.float32) + b1[0])
    o = jnp.dot(o, w2[...].T, preferred_element_type=jnp.float32) + b2[0]
    out_ref[...] = p * jnp.exp(o)


def _readout_call(x4, o_x, linr_W, linr_b, lino_W1, lino_b1, lino_W2, lino_b2):
    BB = 32
    grid = (B // BB,)
    return pl.pallas_call(
        _readout_body,
        grid=grid,
        in_specs=[
            pl.BlockSpec((BB, S + R, C), lambda i: (i, 0, 0)),
            pl.BlockSpec((BB, 5), lambda i: (i, 0)),
            pl.BlockSpec((7, S + R), lambda i: (0, 0)),
            pl.BlockSpec((1, 7), lambda i: (0, 0)),
            pl.BlockSpec((C, 5), lambda i: (0, 0)),
            pl.BlockSpec((1, C), lambda i: (0, 0)),
            pl.BlockSpec((7, C), lambda i: (0, 0)),
            pl.BlockSpec((1, 7), lambda i: (0, 0)),
        ],
        out_specs=pl.BlockSpec((BB, 7), lambda i: (i, 0)),
        out_shape=jax.ShapeDtypeStruct((B, 7), jnp.float32),
    )(x4.reshape(B, S + R, C), o_x, linr_W, linr_b.reshape(1, 7),
      lino_W1, lino_b1.reshape(1, C), lino_W2, lino_b2.reshape(1, 7))


def kernel(s_x, r_x, edge_index, bn_s_w, bn_s_b, bn_r_w, bn_r_b, lin_s_W,
           lin_r_W, conv_W, conv_b, linr_W, linr_b, lino_W1, lino_b1,
           lino_W2, lino_b2):
    src = edge_index[0]
    dst = edge_index[1]
    src2 = jnp.concatenate([src, src + N])

    degp = _deg_call(dst)
    s_emb, r_emb, o_x = _prologue_call(
        s_x, r_x, bn_s_w, bn_s_b, bn_r_w, bn_r_b, lin_s_W, lin_r_W)
    x = jnp.concatenate(
        [s_emb.reshape(B, S, C), r_emb.reshape(B, R, C)], axis=1).reshape(N, C)

    dinv, h2 = _dense0_call(x, degp.reshape(2, N, 1), conv_W[0])
    for l in range(NUM_LAYERS):
        agg2 = _agg_call(src2, dst, h2.reshape(2 * N, 16))
        if l < NUM_LAYERS - 1:
            x, h2 = _dense_call(x, agg2.reshape(2, N, 16), dinv,
                                conv_b[l], conv_W[l + 1])
        else:
            x = _dense_last_call(x, agg2.reshape(2, N, 16), dinv, conv_b[l])

    return _readout_call(x, o_x, linr_W, linr_b,
                         lino_W1, lino_b1, lino_W2, lino_b2)
